# baseline (speedup 1.0000x reference)
import functools

import jax
import jax.numpy as jnp
from jax import lax
from jax.experimental import pallas as pl
from jax.experimental.pallas import tpu as pltpu
from jax.experimental.pallas import tpu_sc as plsc

DIM = 128
NC = 2
NS = 16
NW = NC * NS
CHUNK = 128


NBUF = 6
LOOKAHEAD = 3


def _gather_body(table, sidx, tidx, out_s, out_t, idx_v, rows_v, gsem, wsem):
    wid = lax.axis_index("s") * NC + lax.axis_index("c")
    n_rows_idx = sidx.shape[0]
    rows_per_w = n_rows_idx // NW
    per_w = rows_per_w * CHUNK
    base = wid * rows_per_w
    nch = 2 * rows_per_w

    pltpu.sync_copy(sidx.at[pl.ds(base, rows_per_w)],
                    idx_v.at[pl.ds(0, rows_per_w)])
    pltpu.sync_copy(tidx.at[pl.ds(base, rows_per_w)],
                    idx_v.at[pl.ds(rows_per_w, rows_per_w)])

    def buf(k):
        return rows_v.at[pl.ds((k % NBUF) * CHUNK, CHUNK)]

    def out_slice(k):
        out = out_s if k < rows_per_w else out_t
        c = k % rows_per_w
        return out.at[pl.ds(wid * per_w + c * CHUNK, CHUNK)]

    def fire_gather(k):
        return pltpu.async_copy(table.at[idx_v.at[k]], buf(k), gsem)

    gh = [None] * nch
    wh = [None] * nch
    w_drained = [False] * nch
    for f in range(min(LOOKAHEAD, nch)):
        gh[f] = fire_gather(f)
    for k in range(nch):
        f = k + LOOKAHEAD
        if f < nch:
            if f >= NBUF:
                wh[f - NBUF].wait()
                w_drained[f - NBUF] = True
            gh[f] = fire_gather(f)
        gh[k].wait()
        wh[k] = pltpu.async_copy(buf(k), out_slice(k), wsem)
    for k in range(nch):
        if not w_drained[k]:
            wh[k].wait()


def _sc_gather(Eemb, sidx, tidx):
    B = sidx.shape[0] * CHUNK
    rows_per_w = sidx.shape[0] // NW
    mesh = plsc.VectorSubcoreMesh(core_axis_name="c", subcore_axis_name="s")
    f = functools.partial(
        pl.kernel,
        mesh=mesh,
        out_type=[
            jax.ShapeDtypeStruct((B, DIM), jnp.float32),
            jax.ShapeDtypeStruct((B, DIM), jnp.float32),
        ],
        scratch_types=[
            pltpu.VMEM((2 * rows_per_w, CHUNK), jnp.int32),
            pltpu.VMEM((NBUF * CHUNK, DIM), jnp.float32),
            pltpu.SemaphoreType.DMA,
            pltpu.SemaphoreType.DMA,
        ],
    )(_gather_body)
    return f(Eemb, sidx, tidx)


def _tc_body(sr_ref, tr_ref, sw_ref, tw_ref, w_ref, so_ref, to_ref):
    blk = sr_ref.shape[0]
    w = w_ref[:]
    x = sr_ref[:]
    ns = jnp.sum(x * x, axis=1, keepdims=True)
    cs = sw_ref[:].reshape(blk, 1) * jnp.where(ns > 1.0, lax.rsqrt(ns), 1.0)
    so_ref[:] = jnp.dot(x, w, preferred_element_type=jnp.float32) * cs
    y = tr_ref[:]
    nt = jnp.sum(y * y, axis=1, keepdims=True)
    ct = tw_ref[:].reshape(blk, 1) * jnp.where(nt > 1.0, lax.rsqrt(nt), 1.0)
    to_ref[:] = jnp.dot(y, w, preferred_element_type=jnp.float32) * ct


def _tc_apply(s_rows, t_rows, s_w, t_w, W, blk=2048):
    B = s_rows.shape[0]
    grid = (B // blk,)
    row_spec = pl.BlockSpec((blk, DIM), lambda i: (i, 0))
    w_spec = pl.BlockSpec((blk,), lambda i: (i,))
    return pl.pallas_call(
        _tc_body,
        grid=grid,
        in_specs=[row_spec, row_spec, w_spec, w_spec,
                  pl.BlockSpec((DIM, DIM), lambda i: (0, 0))],
        out_specs=[row_spec, row_spec],
        out_shape=[jax.ShapeDtypeStruct((B, DIM), jnp.float32)] * 2,
        compiler_params=pltpu.CompilerParams(
            dimension_semantics=("parallel",)),
    )(s_rows, t_rows, s_w, t_w, W)


def kernel(S_in, T_in, anc, s_weight, t_weight, Eemb, W):
    B = S_in.shape[0]
    sidx = S_in.astype(jnp.int32).reshape(B // CHUNK, CHUNK)
    tidx = T_in.astype(jnp.int32).reshape(B // CHUNK, CHUNK)
    s_rows, t_rows = _sc_gather(Eemb, sidx, tidx)
    S_out, T_out = _tc_apply(s_rows, t_rows, s_weight, t_weight, W)
    return (S_out, T_out)


# speedup vs baseline: 1.0182x; 1.0182x over previous
import functools

import jax
import jax.numpy as jnp
from jax import lax
from jax.experimental import pallas as pl
from jax.experimental.pallas import tpu as pltpu
from jax.experimental.pallas import tpu_sc as plsc

DIM = 128
NC = 2
NS = 16
NW = NC * NS
CHUNK = 128


NBUF = 6
LOOKAHEAD = 3


def _gather_body(table, sidx, tidx, out_s, out_t, idx_v, rows_v, gsem, wsem):
    wid = lax.axis_index("s") * NC + lax.axis_index("c")
    n_rows_idx = sidx.shape[0]
    rows_per_w = n_rows_idx // NW
    per_w = rows_per_w * CHUNK
    base = wid * rows_per_w
    nch = 2 * rows_per_w

    pltpu.sync_copy(sidx.at[pl.ds(base, rows_per_w)],
                    idx_v.at[pl.ds(0, rows_per_w)])
    pltpu.sync_copy(tidx.at[pl.ds(base, rows_per_w)],
                    idx_v.at[pl.ds(rows_per_w, rows_per_w)])

    def buf(k):
        return rows_v.at[pl.ds((k % NBUF) * CHUNK, CHUNK)]

    def out_slice(k):
        out = out_s if k < rows_per_w else out_t
        c = k % rows_per_w
        return out.at[pl.ds(wid * per_w + c * CHUNK, CHUNK)]

    def fire_gather(k):
        return pltpu.async_copy(table.at[idx_v.at[k]], buf(k), gsem)

    gh = [None] * nch
    wh = [None] * nch
    w_drained = [False] * nch
    for f in range(min(LOOKAHEAD, nch)):
        gh[f] = fire_gather(f)
    for k in range(nch):
        f = k + LOOKAHEAD
        if f < nch:
            if f >= NBUF:
                wh[f - NBUF].wait()
                w_drained[f - NBUF] = True
            gh[f] = fire_gather(f)
        gh[k].wait()
        wh[k] = pltpu.async_copy(buf(k), out_slice(k), wsem)
    for k in range(nch):
        if not w_drained[k]:
            wh[k].wait()


def _sc_gather(Eemb, sidx, tidx):
    B = sidx.shape[0] * CHUNK
    rows_per_w = sidx.shape[0] // NW
    mesh = plsc.VectorSubcoreMesh(core_axis_name="c", subcore_axis_name="s")
    f = functools.partial(
        pl.kernel,
        mesh=mesh,
        out_type=[
            jax.ShapeDtypeStruct((B, DIM), jnp.float32),
            jax.ShapeDtypeStruct((B, DIM), jnp.float32),
        ],
        scratch_types=[
            pltpu.VMEM((2 * rows_per_w, CHUNK), jnp.int32),
            pltpu.VMEM((NBUF * CHUNK, DIM), jnp.float32),
            pltpu.SemaphoreType.DMA,
            pltpu.SemaphoreType.DMA,
        ],
    )(_gather_body)
    return f(Eemb, sidx, tidx)


def _tc_body(sr_ref, tr_ref, sw_ref, tw_ref, w_ref, so_ref, to_ref):
    blk = sr_ref.shape[0]
    w = w_ref[:]
    x = sr_ref[:]
    ns = jnp.sum(x * x, axis=1, keepdims=True)
    cs = sw_ref[:].reshape(blk, 1) * jnp.where(ns > 1.0, lax.rsqrt(ns), 1.0)
    so_ref[:] = jnp.dot(x, w, preferred_element_type=jnp.float32) * cs
    y = tr_ref[:]
    nt = jnp.sum(y * y, axis=1, keepdims=True)
    ct = tw_ref[:].reshape(blk, 1) * jnp.where(nt > 1.0, lax.rsqrt(nt), 1.0)
    to_ref[:] = jnp.dot(y, w, preferred_element_type=jnp.float32) * ct


def _tc_apply(s_rows, t_rows, s_w, t_w, W, blk=4096):
    B = s_rows.shape[0]
    grid = (B // blk,)
    row_spec = pl.BlockSpec((blk, DIM), lambda i: (i, 0))
    w_spec = pl.BlockSpec((blk,), lambda i: (i,))
    return pl.pallas_call(
        _tc_body,
        grid=grid,
        in_specs=[row_spec, row_spec, w_spec, w_spec,
                  pl.BlockSpec((DIM, DIM), lambda i: (0, 0))],
        out_specs=[row_spec, row_spec],
        out_shape=[jax.ShapeDtypeStruct((B, DIM), jnp.float32)] * 2,
        compiler_params=pltpu.CompilerParams(
            dimension_semantics=("parallel",)),
    )(s_rows, t_rows, s_w, t_w, W)


def kernel(S_in, T_in, anc, s_weight, t_weight, Eemb, W):
    B = S_in.shape[0]
    sidx = S_in.astype(jnp.int32).reshape(B // CHUNK, CHUNK)
    tidx = T_in.astype(jnp.int32).reshape(B // CHUNK, CHUNK)
    s_rows, t_rows = _sc_gather(Eemb, sidx, tidx)
    S_out, T_out = _tc_apply(s_rows, t_rows, s_weight, t_weight, W)
    return (S_out, T_out)
